# baseline (speedup 1.0000x reference)
import functools

import jax
import jax.numpy as jnp
from jax import lax
from jax.experimental import pallas as pl
from jax.experimental.pallas import tpu as pltpu
from jax.experimental.pallas import tpu_sc as plsc

N = 10000
E = 320000
F_IN = 128
H = 256
D_EMB = 128
G = 64

NPAD = 10240
NC = 2
NS = 16
NW = NC * NS
B = 128
NB = 80
EPAD = NW * NB * B
FAST_C = 1
NB_F = 122
NB_S = 38
RPT = NPAD // NS

_mesh = plsc.VectorSubcoreMesh(
    core_axis_name="c", subcore_axis_name="s", num_cores=NC, num_subcores=NS)


def _deg_body(edges_hbm, zeros_hbm, ones_hbm, out_hbm, eb, ones_v, acc):
    c = lax.axis_index("c")
    s = lax.axis_index("s")
    wid = s * NC + c
    pltpu.sync_copy(zeros_hbm.at[pl.ds(s * RPT, RPT)], acc.at[pl.ds(s * RPT, RPT)])
    pltpu.sync_copy(ones_hbm, ones_v)
    plsc.subcore_barrier()

    def step(i, carry):
        pltpu.sync_copy(edges_hbm.at[wid, i], eb)
        pltpu.sync_copy(ones_v, acc.at[eb.at[1]], add=True)
        return carry

    lax.fori_loop(0, NB_S, step, 0)

    @pl.when(c == FAST_C)
    def _():
        lax.fori_loop(NB_S, NB_F, step, 0)
    plsc.subcore_barrier()
    pltpu.sync_copy(acc.at[pl.ds(s * RPT, RPT)], out_hbm.at[c, pl.ds(s * RPT, RPT)])


_deg_kernel = functools.partial(
    pl.kernel,
    out_type=jax.ShapeDtypeStruct((NC, NPAD, 128), jnp.float32),
    mesh=_mesh,
    scratch_types=[
        pltpu.VMEM((2, B), jnp.int32),
        pltpu.VMEM((B, 128), jnp.float32),
        pltpu.VMEM_SHARED((NPAD, 128), jnp.float32),
    ],
)(_deg_body)


def _aggp_body(edges_hbm, p_hbm, zeros_hbm, out_hbm, eb, rows, acc, gsem):
    c = lax.axis_index("c")
    s = lax.axis_index("s")
    wid = s * NC + c
    pltpu.sync_copy(zeros_hbm.at[pl.ds(s * RPT, RPT)], acc.at[pl.ds(s * RPT, RPT)])
    plsc.subcore_barrier()

    def step(i, carry):
        pltpu.sync_copy(edges_hbm.at[wid, i], eb)
        pltpu.async_copy(p_hbm.at[eb.at[0]], rows, gsem).wait()
        pltpu.sync_copy(rows, acc.at[eb.at[1]], add=True)
        return carry

    lax.fori_loop(0, NB_S, step, 0)

    @pl.when(c == FAST_C)
    def _():
        lax.fori_loop(NB_S, NB_F, step, 0)
    plsc.subcore_barrier()
    pltpu.sync_copy(acc.at[pl.ds(s * RPT, RPT)], out_hbm.at[c, pl.ds(s * RPT, RPT)])


_aggp_kernel = functools.partial(
    pl.kernel,
    out_type=jax.ShapeDtypeStruct((NC, NPAD, 256), jnp.bfloat16),
    mesh=_mesh,
    compiler_params=pltpu.CompilerParams(use_tc_tiling_on_sc=False),
    scratch_types=[
        pltpu.VMEM((2, B), jnp.int32),
        pltpu.VMEM((B, 256), jnp.bfloat16),
        pltpu.VMEM_SHARED((NPAD, 256), jnp.bfloat16),
        pltpu.SemaphoreType.DMA,
    ],
)(_aggp_body)


_agg1_kernel = functools.partial(
    pl.kernel,
    out_type=jax.ShapeDtypeStruct((NC, NPAD, 128), jnp.float32),
    mesh=_mesh,
    scratch_types=[
        pltpu.VMEM((2, B), jnp.int32),
        pltpu.VMEM((B, 128), jnp.float32),
        pltpu.VMEM_SHARED((NPAD, 128), jnp.float32),
        pltpu.SemaphoreType.DMA,
    ],
)(_aggp_body)


_RB = 2560


def _prep_body(deg_ref, x_ref, w1_ref, dinv_ref, p_ref):
    degsum = deg_ref[0] + deg_ref[1]
    deg = degsum[:, 0:1] + 1.0
    dinv = lax.rsqrt(deg)
    dinv_ref[...] = jnp.broadcast_to(dinv, (_RB, 128))
    h = jnp.dot(x_ref[...], w1_ref[...], preferred_element_type=jnp.float32)
    h = h * dinv
    p_ref[...] = h.astype(jnp.bfloat16)


def _prep(deg, x_pad, W1):
    row_spec = pl.BlockSpec((_RB, 128), lambda i: (i, 0))
    return pl.pallas_call(
        _prep_body,
        grid=(NPAD // _RB,),
        in_specs=[
            pl.BlockSpec((NC, _RB, 128), lambda i: (0, i, 0)),
            row_spec,
            pl.BlockSpec((F_IN, H), lambda i: (0, 0)),
        ],
        out_specs=[row_spec, pl.BlockSpec((_RB, H), lambda i: (i, 0))],
        out_shape=[
            jax.ShapeDtypeStruct((NPAD, 128), jnp.float32),
            jax.ShapeDtypeStruct((NPAD, H), jnp.bfloat16),
        ],
    )(deg, x_pad, W1)


def _combine2_body(a_ref, p_ref, dinv_ref, b_ref, w_ref, q_ref):
    dinv = dinv_ref[...]
    agg = (a_ref[0].astype(jnp.float32) + a_ref[1].astype(jnp.float32)
           + p_ref[...].astype(jnp.float32))
    h = agg * dinv[:, 0:1] + b_ref[...]
    h = jnp.maximum(h, 0.0)
    q = jnp.dot(h, w_ref[...], preferred_element_type=jnp.float32) * dinv[:, 0:1]
    q_ref[...] = q.astype(jnp.bfloat16)


def _combine3_body(a_ref, p_ref, dinv_ref, b_ref, w_ref, q_ref):
    dinv = dinv_ref[...]
    agg = (a_ref[0].astype(jnp.float32) + a_ref[1].astype(jnp.float32)
           + p_ref[...].astype(jnp.float32))
    h = agg * dinv[:, 0:1] + b_ref[...]
    h = jnp.maximum(h, 0.0)
    q_ref[...] = jnp.dot(h, w_ref[...], preferred_element_type=jnp.float32) * dinv


def _combine(a, p, dinv_b, b_vec, W, body, out_shape):
    grid = (NPAD // _RB,)
    w_last = W.shape[1]
    row_spec = pl.BlockSpec((_RB, 128), lambda i: (i, 0))
    pk_spec = pl.BlockSpec((_RB, H), lambda i: (i, 0))
    out_spec = pk_spec if out_shape[0].dtype == jnp.bfloat16 else row_spec
    return pl.pallas_call(
        body,
        grid=grid,
        in_specs=[
            pl.BlockSpec((NC, _RB, H), lambda i: (0, i, 0)),
            pk_spec, row_spec,
            pl.BlockSpec((1, H), lambda i: (0, 0)),
            pl.BlockSpec((H, w_last), lambda i: (0, 0)),
        ],
        out_specs=[out_spec],
        out_shape=out_shape,
    )(a, p, dinv_b, b_vec, W)


def _final_body(a_ref, p_ref, dinv_ref, b_ref, batch_ref, out_ref):
    h = (a_ref[0] + a_ref[1] + p_ref[...]) * dinv_ref[...] + b_ref[...]
    bvec = batch_ref[...]
    seg = lax.broadcasted_iota(jnp.int32, (G, NPAD), 0)
    m = (jnp.broadcast_to(bvec, (G, NPAD)) == seg).astype(jnp.float32)
    summ = jnp.dot(m, h, preferred_element_type=jnp.float32)
    cnt = jnp.sum(m, axis=1, keepdims=True)
    pooled = summ / jnp.maximum(cnt, 1.0)
    nrm = jnp.sqrt(jnp.sum(pooled * pooled, axis=1, keepdims=True))
    out_ref[...] = pooled / jnp.maximum(nrm, 1e-12)


def _final(a, p, dinv_b, b_vec, batch_2d):
    return pl.pallas_call(
        _final_body,
        out_shape=jax.ShapeDtypeStruct((G, D_EMB), jnp.float32),
    )(a, p, dinv_b, b_vec, batch_2d)


def kernel(x, edge_index, batch, W1, b1, W2, b2, W3, b3):
    f32 = jnp.float32
    bf16 = jnp.bfloat16
    i32 = jnp.int32
    pad_e = EPAD - E
    src = jnp.concatenate([edge_index[0], jnp.full((pad_e,), NPAD - 1, i32)])
    dst = jnp.concatenate([edge_index[1], jnp.full((pad_e,), NPAD - 1, i32)])
    es = jnp.stack([src, dst])
    nf = NS * NB_F * B
    fast = es[:, :nf].reshape(2, NS, NB_F, B).transpose(1, 2, 0, 3)
    slow = es[:, nf:].reshape(2, NS, NB_S, B).transpose(1, 2, 0, 3)
    slow = jnp.concatenate(
        [slow, jnp.full((NS, NB_F - NB_S, 2, B), NPAD - 1, i32)], axis=1)
    per_core = [slow, fast] if FAST_C == 1 else [fast, slow]
    edges_w = jnp.stack(per_core, axis=1).reshape(NW, NB_F, 2, B)
    x_pad = jnp.concatenate([x, jnp.zeros((NPAD - N, F_IN), f32)], axis=0)
    batch_2d = jnp.concatenate([batch, jnp.full((NPAD - N,), G, i32)])[None, :]
    ones128 = jnp.ones((B, 128), f32)
    zeros128 = jnp.zeros((NPAD, 128), f32)
    zeros_pk = jnp.zeros((NPAD, H), bf16)

    deg = _deg_kernel(edges_w, zeros128, ones128)
    dinv_b, p1 = _prep(deg, x_pad, W1)

    a = _aggp_kernel(edges_w, p1, zeros_pk)
    (p2,) = _combine(
        a, p1, dinv_b, b1[None, :], W2, _combine2_body,
        [jax.ShapeDtypeStruct((NPAD, H), bf16)])

    a = _aggp_kernel(edges_w, p2, zeros_pk)
    (r0,) = _combine(
        a, p2, dinv_b, b2[None, :], W3, _combine3_body,
        [jax.ShapeDtypeStruct((NPAD, 128), f32)])

    a1 = _agg1_kernel(edges_w, r0, zeros128)
    return _final(a1, r0, dinv_b, b3[None, :], batch_2d)


# speedup vs baseline: 1.0147x; 1.0147x over previous
import functools

import jax
import jax.numpy as jnp
from jax import lax
from jax.experimental import pallas as pl
from jax.experimental.pallas import tpu as pltpu
from jax.experimental.pallas import tpu_sc as plsc

N = 10000
E = 320000
F_IN = 128
H = 256
D_EMB = 128
G = 64

NPAD = 10240
NC = 2
NS = 16
NW = NC * NS
B = 128
NB = 80
EPAD = NW * NB * B
FAST_C = 1
NB_F = 114
NB_S = 46
RPT = NPAD // NS

_mesh = plsc.VectorSubcoreMesh(
    core_axis_name="c", subcore_axis_name="s", num_cores=NC, num_subcores=NS)


def _deg_body(edges_hbm, zeros_hbm, ones_hbm, out_hbm, eb, ones_v, acc):
    c = lax.axis_index("c")
    s = lax.axis_index("s")
    wid = s * NC + c
    pltpu.sync_copy(zeros_hbm.at[pl.ds(s * RPT, RPT)], acc.at[pl.ds(s * RPT, RPT)])
    pltpu.sync_copy(ones_hbm, ones_v)
    plsc.subcore_barrier()

    def step(i, carry):
        pltpu.sync_copy(edges_hbm.at[wid, i], eb)
        pltpu.sync_copy(ones_v, acc.at[eb.at[1]], add=True)
        return carry

    lax.fori_loop(0, NB_S, step, 0)

    @pl.when(c == FAST_C)
    def _():
        lax.fori_loop(NB_S, NB_F, step, 0)
    plsc.subcore_barrier()
    pltpu.sync_copy(acc.at[pl.ds(s * RPT, RPT)], out_hbm.at[c, pl.ds(s * RPT, RPT)])


_deg_kernel = functools.partial(
    pl.kernel,
    out_type=jax.ShapeDtypeStruct((NC, NPAD, 128), jnp.float32),
    mesh=_mesh,
    scratch_types=[
        pltpu.VMEM((2, B), jnp.int32),
        pltpu.VMEM((B, 128), jnp.float32),
        pltpu.VMEM_SHARED((NPAD, 128), jnp.float32),
    ],
)(_deg_body)


def _aggp_body(edges_hbm, p_hbm, zeros_hbm, out_hbm, eb, rows, acc, gsem):
    c = lax.axis_index("c")
    s = lax.axis_index("s")
    wid = s * NC + c
    pltpu.sync_copy(zeros_hbm.at[pl.ds(s * RPT, RPT)], acc.at[pl.ds(s * RPT, RPT)])
    plsc.subcore_barrier()

    def step(i, carry):
        pltpu.sync_copy(edges_hbm.at[wid, i], eb)
        pltpu.async_copy(p_hbm.at[eb.at[0]], rows, gsem).wait()
        pltpu.sync_copy(rows, acc.at[eb.at[1]], add=True)
        return carry

    lax.fori_loop(0, NB_S, step, 0)

    @pl.when(c == FAST_C)
    def _():
        lax.fori_loop(NB_S, NB_F, step, 0)
    plsc.subcore_barrier()
    pltpu.sync_copy(acc.at[pl.ds(s * RPT, RPT)], out_hbm.at[c, pl.ds(s * RPT, RPT)])


_aggp_kernel = functools.partial(
    pl.kernel,
    out_type=jax.ShapeDtypeStruct((NC, NPAD, 256), jnp.bfloat16),
    mesh=_mesh,
    compiler_params=pltpu.CompilerParams(use_tc_tiling_on_sc=False),
    scratch_types=[
        pltpu.VMEM((2, B), jnp.int32),
        pltpu.VMEM((B, 256), jnp.bfloat16),
        pltpu.VMEM_SHARED((NPAD, 256), jnp.bfloat16),
        pltpu.SemaphoreType.DMA,
    ],
)(_aggp_body)


_agg1_kernel = functools.partial(
    pl.kernel,
    out_type=jax.ShapeDtypeStruct((NC, NPAD, 128), jnp.float32),
    mesh=_mesh,
    scratch_types=[
        pltpu.VMEM((2, B), jnp.int32),
        pltpu.VMEM((B, 128), jnp.float32),
        pltpu.VMEM_SHARED((NPAD, 128), jnp.float32),
        pltpu.SemaphoreType.DMA,
    ],
)(_aggp_body)


_RB = 2560


def _prep_body(deg_ref, x_ref, w1_ref, dinv_ref, p_ref):
    degsum = deg_ref[0] + deg_ref[1]
    deg = degsum[:, 0:1] + 1.0
    dinv = lax.rsqrt(deg)
    dinv_ref[...] = jnp.broadcast_to(dinv, (_RB, 128))
    h = jnp.dot(x_ref[...], w1_ref[...], preferred_element_type=jnp.float32)
    h = h * dinv
    p_ref[...] = h.astype(jnp.bfloat16)


def _prep(deg, x_pad, W1):
    row_spec = pl.BlockSpec((_RB, 128), lambda i: (i, 0))
    return pl.pallas_call(
        _prep_body,
        grid=(NPAD // _RB,),
        in_specs=[
            pl.BlockSpec((NC, _RB, 128), lambda i: (0, i, 0)),
            row_spec,
            pl.BlockSpec((F_IN, H), lambda i: (0, 0)),
        ],
        out_specs=[row_spec, pl.BlockSpec((_RB, H), lambda i: (i, 0))],
        out_shape=[
            jax.ShapeDtypeStruct((NPAD, 128), jnp.float32),
            jax.ShapeDtypeStruct((NPAD, H), jnp.bfloat16),
        ],
    )(deg, x_pad, W1)


def _combine2_body(a_ref, p_ref, dinv_ref, b_ref, w_ref, q_ref):
    dinv = dinv_ref[...]
    agg = (a_ref[0].astype(jnp.float32) + a_ref[1].astype(jnp.float32)
           + p_ref[...].astype(jnp.float32))
    h = agg * dinv[:, 0:1] + b_ref[...]
    h = jnp.maximum(h, 0.0)
    q = jnp.dot(h, w_ref[...], preferred_element_type=jnp.float32) * dinv[:, 0:1]
    q_ref[...] = q.astype(jnp.bfloat16)


def _combine3_body(a_ref, p_ref, dinv_ref, b_ref, w_ref, q_ref):
    dinv = dinv_ref[...]
    agg = (a_ref[0].astype(jnp.float32) + a_ref[1].astype(jnp.float32)
           + p_ref[...].astype(jnp.float32))
    h = agg * dinv[:, 0:1] + b_ref[...]
    h = jnp.maximum(h, 0.0)
    q_ref[...] = jnp.dot(h, w_ref[...], preferred_element_type=jnp.float32) * dinv


def _combine(a, p, dinv_b, b_vec, W, body, out_shape):
    grid = (NPAD // _RB,)
    w_last = W.shape[1]
    row_spec = pl.BlockSpec((_RB, 128), lambda i: (i, 0))
    pk_spec = pl.BlockSpec((_RB, H), lambda i: (i, 0))
    out_spec = pk_spec if out_shape[0].dtype == jnp.bfloat16 else row_spec
    return pl.pallas_call(
        body,
        grid=grid,
        in_specs=[
            pl.BlockSpec((NC, _RB, H), lambda i: (0, i, 0)),
            pk_spec, row_spec,
            pl.BlockSpec((1, H), lambda i: (0, 0)),
            pl.BlockSpec((H, w_last), lambda i: (0, 0)),
        ],
        out_specs=[out_spec],
        out_shape=out_shape,
    )(a, p, dinv_b, b_vec, W)


def _final_body(a_ref, p_ref, dinv_ref, b_ref, batch_ref, out_ref):
    h = (a_ref[0] + a_ref[1] + p_ref[...]) * dinv_ref[...] + b_ref[...]
    bvec = batch_ref[...]
    seg = lax.broadcasted_iota(jnp.int32, (G, NPAD), 0)
    m = (jnp.broadcast_to(bvec, (G, NPAD)) == seg).astype(jnp.float32)
    summ = jnp.dot(m, h, preferred_element_type=jnp.float32)
    cnt = jnp.sum(m, axis=1, keepdims=True)
    pooled = summ / jnp.maximum(cnt, 1.0)
    nrm = jnp.sqrt(jnp.sum(pooled * pooled, axis=1, keepdims=True))
    out_ref[...] = pooled / jnp.maximum(nrm, 1e-12)


def _final(a, p, dinv_b, b_vec, batch_2d):
    return pl.pallas_call(
        _final_body,
        out_shape=jax.ShapeDtypeStruct((G, D_EMB), jnp.float32),
    )(a, p, dinv_b, b_vec, batch_2d)


def kernel(x, edge_index, batch, W1, b1, W2, b2, W3, b3):
    f32 = jnp.float32
    bf16 = jnp.bfloat16
    i32 = jnp.int32
    pad_e = EPAD - E
    src = jnp.concatenate([edge_index[0], jnp.full((pad_e,), NPAD - 1, i32)])
    dst = jnp.concatenate([edge_index[1], jnp.full((pad_e,), NPAD - 1, i32)])
    es = jnp.stack([src, dst])
    nf = NS * NB_F * B
    fast = es[:, :nf].reshape(2, NS, NB_F, B).transpose(1, 2, 0, 3)
    slow = es[:, nf:].reshape(2, NS, NB_S, B).transpose(1, 2, 0, 3)
    slow = jnp.concatenate(
        [slow, jnp.full((NS, NB_F - NB_S, 2, B), NPAD - 1, i32)], axis=1)
    per_core = [slow, fast] if FAST_C == 1 else [fast, slow]
    edges_w = jnp.stack(per_core, axis=1).reshape(NW, NB_F, 2, B)
    x_pad = jnp.concatenate([x, jnp.zeros((NPAD - N, F_IN), f32)], axis=0)
    batch_2d = jnp.concatenate([batch, jnp.full((NPAD - N,), G, i32)])[None, :]
    ones128 = jnp.ones((B, 128), f32)
    zeros128 = jnp.zeros((NPAD, 128), f32)
    zeros_pk = jnp.zeros((NPAD, H), bf16)

    deg = _deg_kernel(edges_w, zeros128, ones128)
    dinv_b, p1 = _prep(deg, x_pad, W1)

    a = _aggp_kernel(edges_w, p1, zeros_pk)
    (p2,) = _combine(
        a, p1, dinv_b, b1[None, :], W2, _combine2_body,
        [jax.ShapeDtypeStruct((NPAD, H), bf16)])

    a = _aggp_kernel(edges_w, p2, zeros_pk)
    (r0,) = _combine(
        a, p2, dinv_b, b2[None, :], W3, _combine3_body,
        [jax.ShapeDtypeStruct((NPAD, 128), f32)])

    a1 = _agg1_kernel(edges_w, r0, zeros128)
    return _final(a1, r0, dinv_b, b3[None, :], batch_2d)
